# baseline (speedup 1.0000x reference)
import jax
import jax.numpy as jnp
from jax import lax
from jax.experimental import pallas as pl
from jax.experimental.pallas import tpu as pltpu
from jax.experimental.pallas import tpu_sc as plsc

NC = 2
NS = 16
NW = NC * NS

B, S, H = 4096, 200, 64
BPW = B // NW
HQ = H // 16
NBUF = 3
NGRP = S // NBUF


def _body(hid_hbm, iid_hbm, sid_hbm, itab_hbm, stab_hbm, out_hbm,
          ids_v, iid_v, instR_v, h0, h1, h2, r0, r1, r2,
          si0, si1, si2, so0, so1, so2, gsem):
    hbufs = (h0, h1, h2)
    rbufs = (r0, r1, r2)
    sem_in = (si0, si1, si2)
    sem_out = (so0, so1, so2)

    cid = lax.axis_index("c")
    sid = lax.axis_index("s")
    wid = sid * NC + cid
    base = wid * BPW

    pltpu.sync_copy(sid_hbm.at[:, pl.ds(base, BPW)], ids_v)
    pltpu.sync_copy(iid_hbm.at[pl.ds(base, BPW)], iid_v)
    pltpu.async_copy(itab_hbm.at[iid_v], instR_v, gsem).wait()

    def in_copies(s, k):
        return (
            pltpu.make_async_copy(hid_hbm.at[s, :, pl.ds(base, BPW)],
                                  hbufs[k], sem_in[k]),
            pltpu.make_async_copy(stab_hbm.at[ids_v.at[s]],
                                  rbufs[k], sem_in[k]),
        )

    def fire_in(s, k):
        for c in in_copies(s, k):
            c.start()

    def wait_in(s, k):
        for c in in_copies(s, k):
            c.wait()

    def out_copy(s, k):
        return pltpu.make_async_copy(hbufs[k],
                                     out_hbm.at[s, :, pl.ds(base, BPW)],
                                     sem_out[k])

    idx_h = [lax.iota(jnp.int32, 16) + 16 * q for q in range(HQ)]

    def compute(s, k):
        hb = hbufs[k]
        rb = rbufs[k]

        def lane(l, c):
            ls = jnp.full((16,), 0, jnp.int32) + l
            for q in range(HQ):
                val = rb[l, pl.ds(16 * q, 16)] + instR_v[l, pl.ds(16 * q, 16)]
                plsc.addupdate_scatter(hb, [idx_h[q], ls], val)
            return c

        lax.fori_loop(0, BPW, lane, 0, unroll=2)

    fire_in(0, 0)
    fire_in(1, 1)

    wait_in(0, 0)
    compute(0, 0)
    out_copy(0, 0).start()
    fire_in(2, 2)

    wait_in(1, 1)
    compute(1, 1)
    out_copy(1, 1).start()
    out_copy(0, 0).wait()
    fire_in(3, 0)

    wait_in(2, 2)
    compute(2, 2)
    out_copy(2, 2).start()
    out_copy(1, 1).wait()
    fire_in(4, 1)

    def group(g, carry):
        for b in range(NBUF):
            s = NBUF * g + b
            k = b
            k2 = (b + 2) % NBUF
            wait_in(s, k)
            compute(s, k)
            out_copy(s, k).start()
            out_copy(s - 1, k2).wait()
            fire_in(s + 2, k2)
        return carry

    lax.fori_loop(1, NGRP, group, 0)

    s = NBUF * NGRP
    wait_in(s, 0)
    compute(s, 0)
    out_copy(s, 0).start()

    wait_in(s + 1, 1)
    compute(s + 1, 1)
    out_copy(s + 1, 1).start()

    out_copy(s - 1, 2).wait()
    out_copy(s, 0).wait()
    out_copy(s + 1, 1).wait()


def kernel(hidden_states, instrument_ids, session_ids, instrument_table,
           session_table):
    hid_t = jnp.transpose(hidden_states, (1, 2, 0))
    sid_t = jnp.transpose(session_ids.astype(jnp.int32), (1, 0))

    k = pl.kernel(
        _body,
        out_type=jax.ShapeDtypeStruct((S, H, B), jnp.float32),
        mesh=plsc.VectorSubcoreMesh(core_axis_name="c", subcore_axis_name="s",
                                    num_cores=NC, num_subcores=NS),
        compiler_params=pltpu.CompilerParams(use_tc_tiling_on_sc=False,
                                             needs_layout_passes=False),
        scratch_types=(
            [pltpu.VMEM((S, BPW), jnp.int32),
             pltpu.VMEM((BPW,), jnp.int32),
             pltpu.VMEM((BPW, H), jnp.float32)]
            + [pltpu.VMEM((H, BPW), jnp.float32) for _ in range(NBUF)]
            + [pltpu.VMEM((BPW, H), jnp.float32) for _ in range(NBUF)]
            + [pltpu.SemaphoreType.DMA for _ in range(2 * NBUF + 1)]
        ),
    )
    out_t = k(hid_t, instrument_ids.astype(jnp.int32), sid_t,
              instrument_table, session_table)
    return jnp.transpose(out_t, (2, 0, 1))


# speedup vs baseline: 2.2089x; 2.2089x over previous
import jax
import jax.numpy as jnp
from jax import lax
from jax.experimental import pallas as pl
from jax.experimental.pallas import tpu as pltpu
from jax.experimental.pallas import tpu_sc as plsc

NC = 2
NS = 16
NW = NC * NS

B, S, H = 4096, 200, 64
NHG = 4
NBB = NW // NHG
BPW = B // NBB
NG = BPW // 16
NBUF = 3
NGRP = S // NBUF


def _body(hid_hbm, iid_hbm, sid_hbm, itab_hbm, stab_hbm, out_hbm,
          stab_v, inst_v, instR_v, iid_v, rottab_v,
          h0, h1, h2, i0, i1, i2,
          si0, si1, si2, so0, so1, so2, gsem):
    hbufs = (h0, h1, h2)
    ibufs = (i0, i1, i2)
    sem_in = (si0, si1, si2)
    sem_out = (so0, so1, so2)

    cid = lax.axis_index("c")
    scid = lax.axis_index("s")
    wid = scid * NC + cid
    hg = wid // NBB
    bb = wid % NBB
    hbase = hg * 16
    bbase = bb * BPW

    iota = lax.iota(jnp.int32, 16)

    for d in range(16):
        rottab_v[d, :] = (iota + d) & 15

    pltpu.sync_copy(stab_hbm, stab_v)
    pltpu.sync_copy(iid_hbm.at[pl.ds(bbase, BPW)], iid_v)
    for r in range(BPW // 128):
        pltpu.async_copy(itab_hbm.at[iid_v.at[pl.ds(128 * r, 128)]],
                         instR_v, gsem).wait()
        def tr_d(d, c):
            rot = (iota + d) & 15
            for g in range(8):
                col = iota + (16 * g + 128 * r)
                v = plsc.load_gather(instR_v, [iota + 16 * g, rot + hbase])
                plsc.store_scatter(inst_v, [rot, col], v)
            return c

        lax.fori_loop(0, 16, tr_d, 0)

    def in_copies(s, k):
        return (
            pltpu.make_async_copy(
                hid_hbm.at[s, pl.ds(hbase, 16), pl.ds(bbase, BPW)],
                hbufs[k], sem_in[k]),
            pltpu.make_async_copy(sid_hbm.at[s, pl.ds(bbase, BPW)],
                                  ibufs[k], sem_in[k]),
        )

    def fire_in(s, k):
        for c in in_copies(s, k):
            c.start()

    def wait_in(s, k):
        for c in in_copies(s, k):
            c.wait()

    def out_copy(s, k):
        return pltpu.make_async_copy(
            hbufs[k], out_hbm.at[s, pl.ds(hbase, 16), pl.ds(bbase, BPW)],
            sem_out[k])

    def compute(s, k):
        hb = hbufs[k]
        ids_ref = ibufs[k]

        @plsc.parallel_loop(0, 16)
        def rows(r):
            for cix in range(NG):
                plsc.addupdate(hb.at[r, pl.ds(16 * cix, 16)],
                               inst_v[r, pl.ds(16 * cix, 16)])

        @plsc.parallel_loop(0, NG)
        def gloop(g):
            ids_g = ids_ref[pl.ds(16 * g, 16)]
            col = iota + 16 * g
            for d in range(16):
                rot = rottab_v[d, :]
                sval = plsc.load_gather(stab_v, [ids_g, rot + hbase])
                plsc.addupdate_scatter(hb, [rot, col], sval)

    fire_in(0, 0)
    fire_in(1, 1)

    wait_in(0, 0)
    compute(0, 0)
    out_copy(0, 0).start()
    fire_in(2, 2)

    wait_in(1, 1)
    compute(1, 1)
    out_copy(1, 1).start()
    out_copy(0, 0).wait()
    fire_in(3, 0)

    wait_in(2, 2)
    compute(2, 2)
    out_copy(2, 2).start()
    out_copy(1, 1).wait()
    fire_in(4, 1)

    def group(g, carry):
        for b in range(NBUF):
            s = NBUF * g + b
            k = b
            k2 = (b + 2) % NBUF
            wait_in(s, k)
            compute(s, k)
            out_copy(s, k).start()
            out_copy(s - 1, k2).wait()
            fire_in(s + 2, k2)
        return carry

    lax.fori_loop(1, NGRP, group, 0)

    s = NBUF * NGRP
    wait_in(s, 0)
    compute(s, 0)
    out_copy(s, 0).start()

    wait_in(s + 1, 1)
    compute(s + 1, 1)
    out_copy(s + 1, 1).start()

    out_copy(s - 1, 2).wait()
    out_copy(s, 0).wait()
    out_copy(s + 1, 1).wait()


def kernel(hidden_states, instrument_ids, session_ids, instrument_table,
           session_table):
    hid_t = jnp.transpose(hidden_states, (1, 2, 0))
    sid_t = jnp.transpose(session_ids.astype(jnp.int32), (1, 0))

    k = pl.kernel(
        _body,
        out_type=jax.ShapeDtypeStruct((S, H, B), jnp.float32),
        mesh=plsc.VectorSubcoreMesh(core_axis_name="c", subcore_axis_name="s",
                                    num_cores=NC, num_subcores=NS),
        compiler_params=pltpu.CompilerParams(use_tc_tiling_on_sc=False,
                                             needs_layout_passes=False),
        scratch_types=(
            [pltpu.VMEM((1000, H), jnp.float32),
             pltpu.VMEM((16, BPW), jnp.float32),
             pltpu.VMEM((128, H), jnp.float32),
             pltpu.VMEM((BPW,), jnp.int32),
             pltpu.VMEM((16, 16), jnp.int32)]
            + [pltpu.VMEM((16, BPW), jnp.float32) for _ in range(NBUF)]
            + [pltpu.VMEM((BPW,), jnp.int32) for _ in range(NBUF)]
            + [pltpu.SemaphoreType.DMA for _ in range(2 * NBUF + 1)]
        ),
    )
    out_t = k(hid_t, instrument_ids.astype(jnp.int32), sid_t,
              instrument_table, session_table)
    return jnp.transpose(out_t, (2, 0, 1))


# speedup vs baseline: 2.2528x; 1.0199x over previous
import jax
import jax.numpy as jnp
from jax import lax
from jax.experimental import pallas as pl
from jax.experimental.pallas import tpu as pltpu
from jax.experimental.pallas import tpu_sc as plsc

NC = 2
NS = 16
NW = NC * NS

B, S, H = 4096, 200, 64
NHG = 4
NBB = NW // NHG
BPW = B // NBB
NG = BPW // 16
NBUF = 3
NGRP = S // NBUF


def _body(hid_hbm, iid_hbm, sid_hbm, itab_hbm, stab_hbm, out_hbm,
          stab_v, inst_v, iid_v, rottab_v,
          h0, h1, h2, i0, i1, i2,
          si0, si1, si2, so0, so1, so2, gsem):
    hbufs = (h0, h1, h2)
    ibufs = (i0, i1, i2)
    sem_in = (si0, si1, si2)
    sem_out = (so0, so1, so2)

    cid = lax.axis_index("c")
    scid = lax.axis_index("s")
    wid = scid * NC + cid
    hg = wid // NBB
    bb = wid % NBB
    hbase = hg * 16
    bbase = bb * BPW

    iota = lax.iota(jnp.int32, 16)

    for d in range(16):
        rottab_v[d, :] = (iota + d) & 15

    pltpu.sync_copy(stab_hbm, stab_v)
    pltpu.sync_copy(iid_hbm.at[pl.ds(bbase, BPW)], iid_v)
    cps = [pltpu.make_async_copy(
               itab_hbm.at[hbase + h].at[iid_v.at[pl.ds(128 * r, 128)]],
               inst_v.at[h, pl.ds(128 * r, 128)], gsem)
           for h in range(16) for r in range(BPW // 128)]
    for c in cps:
        c.start()
    for c in cps:
        c.wait()

    def in_copies(s, k):
        return (
            pltpu.make_async_copy(
                hid_hbm.at[s, pl.ds(hbase, 16), pl.ds(bbase, BPW)],
                hbufs[k], sem_in[k]),
            pltpu.make_async_copy(sid_hbm.at[s, pl.ds(bbase, BPW)],
                                  ibufs[k], sem_in[k]),
        )

    def fire_in(s, k):
        for c in in_copies(s, k):
            c.start()

    def wait_in(s, k):
        for c in in_copies(s, k):
            c.wait()

    def out_copy(s, k):
        return pltpu.make_async_copy(
            hbufs[k], out_hbm.at[s, pl.ds(hbase, 16), pl.ds(bbase, BPW)],
            sem_out[k])

    def compute(s, k):
        hb = hbufs[k]
        ids_ref = ibufs[k]

        @plsc.parallel_loop(0, 16)
        def rows(r):
            for cix in range(NG):
                plsc.addupdate(hb.at[r, pl.ds(16 * cix, 16)],
                               inst_v[r, pl.ds(16 * cix, 16)])

        @plsc.parallel_loop(0, NG)
        def gloop(g):
            ids_g = ids_ref[pl.ds(16 * g, 16)]
            col = iota + 16 * g
            for d in range(16):
                rot = rottab_v[d, :]
                sval = plsc.load_gather(stab_v, [ids_g, rot + hbase])
                plsc.addupdate_scatter(hb, [rot, col], sval)

    fire_in(0, 0)
    fire_in(1, 1)

    wait_in(0, 0)
    compute(0, 0)
    out_copy(0, 0).start()
    fire_in(2, 2)

    wait_in(1, 1)
    compute(1, 1)
    out_copy(1, 1).start()
    out_copy(0, 0).wait()
    fire_in(3, 0)

    wait_in(2, 2)
    compute(2, 2)
    out_copy(2, 2).start()
    out_copy(1, 1).wait()
    fire_in(4, 1)

    def group(g, carry):
        for b in range(NBUF):
            s = NBUF * g + b
            k = b
            k2 = (b + 2) % NBUF
            wait_in(s, k)
            compute(s, k)
            out_copy(s, k).start()
            out_copy(s - 1, k2).wait()
            fire_in(s + 2, k2)
        return carry

    lax.fori_loop(1, NGRP, group, 0)

    s = NBUF * NGRP
    wait_in(s, 0)
    compute(s, 0)
    out_copy(s, 0).start()

    wait_in(s + 1, 1)
    compute(s + 1, 1)
    out_copy(s + 1, 1).start()

    out_copy(s - 1, 2).wait()
    out_copy(s, 0).wait()
    out_copy(s + 1, 1).wait()


def kernel(hidden_states, instrument_ids, session_ids, instrument_table,
           session_table):
    hid_t = jnp.transpose(hidden_states, (1, 2, 0))
    sid_t = jnp.transpose(session_ids.astype(jnp.int32), (1, 0))
    itab_t = jnp.pad(jnp.transpose(instrument_table, (1, 0)), ((0, 0), (0, 96)))

    k = pl.kernel(
        _body,
        out_type=jax.ShapeDtypeStruct((S, H, B), jnp.float32),
        mesh=plsc.VectorSubcoreMesh(core_axis_name="c", subcore_axis_name="s",
                                    num_cores=NC, num_subcores=NS),
        compiler_params=pltpu.CompilerParams(use_tc_tiling_on_sc=False,
                                             needs_layout_passes=False),
        scratch_types=(
            [pltpu.VMEM((1000, H), jnp.float32),
             pltpu.VMEM((16, BPW), jnp.float32),
             pltpu.VMEM((BPW,), jnp.int32),
             pltpu.VMEM((16, 16), jnp.int32)]
            + [pltpu.VMEM((16, BPW), jnp.float32) for _ in range(NBUF)]
            + [pltpu.VMEM((BPW,), jnp.int32) for _ in range(NBUF)]
            + [pltpu.SemaphoreType.DMA for _ in range(2 * NBUF + 1)]
        ),
    )
    out_t = k(hid_t, instrument_ids.astype(jnp.int32), sid_t,
              itab_t, session_table)
    return jnp.transpose(out_t, (2, 0, 1))
